# baseline (speedup 1.0000x reference)
import functools

import jax
import jax.numpy as jnp
from jax import lax
from jax.experimental import pallas as pl
from jax.experimental.pallas import tpu as pltpu

_TAGSET = 45
_BB = 128
_UNROLL = 16
_WC = 16


def _tagger_kernel(idx_ref, xg_tab_ref, whh_ref, wout_ref, bout_ref,
                   out_ref, xg_vmem, sems, *, seq_len, hidden_dim,
                   block_b, n_tags):
    T, H, BB = seq_len, hidden_dim, block_b
    wc = min(_WC, T)
    starts = [0, 16, 24, 28] if T == 32 else list(range(0, T, wc))
    sizes = [b - a for a, b in zip(starts, starts[1:] + [T])]
    nb = pl.program_id(0)
    rows = T * BB

    def issue(k, carry):
        base = k * _UNROLL
        s = base * 0
        for b in starts[1:]:
            s = s + (base >= b * BB).astype(base.dtype)
        for u in range(_UNROLL):
            j = base + u
            pltpu.make_async_copy(
                xg_tab_ref.at[pl.ds(idx_ref[nb, j], 1), :],
                xg_vmem.at[pl.ds(j, 1), :],
                sems.at[s]).start()
        return carry

    lax.fori_loop(0, rows // _UNROLL, issue, 0)

    whh = whh_ref[...]
    wout = wout_ref[...]
    bout = bout_ref[...]

    h = jnp.zeros((BB, H), jnp.float32)
    c = jnp.zeros((BB, H), jnp.float32)
    for t in range(T):
        if t in starts:
            ci = starts.index(t)
            pltpu.make_async_copy(
                xg_tab_ref.at[pl.ds(0, BB * sizes[ci]), :],
                xg_vmem.at[pl.ds(t * BB, BB * sizes[ci]), :],
                sems.at[ci]).wait()
        xg_t = xg_vmem[pl.ds(t * BB, BB), :]
        if t == 0:
            gates = xg_t
        else:
            gates = xg_t + jnp.dot(h, whh,
                                   preferred_element_type=jnp.float32)
        th = jnp.tanh(0.5 * gates)
        i_g = 0.5 * th[:, 0 * H:1 * H] + 0.5
        f_g = 0.5 * th[:, 1 * H:2 * H] + 0.5
        g_g = th[:, 2 * H:3 * H]
        o_g = 0.5 * th[:, 3 * H:4 * H] + 0.5
        c = f_g * c + i_g * g_g
        h = o_g * jnp.tanh(c)
        logits = jnp.dot(h, wout, preferred_element_type=jnp.float32) + bout
        m = jnp.max(logits, axis=1, keepdims=True)
        z = logits - m
        lse = jnp.log(jnp.sum(jnp.exp(z), axis=1, keepdims=True))
        res = (z - lse)[:, :n_tags]
        out_ref[t:t + BB * T:T, :] = res


def kernel(sentences, xg_table, whh, wout, bout):
    B, T = sentences.shape
    H = whh.shape[0]
    VPAD = wout.shape[1]
    BB = _BB if B % _BB == 0 else B
    NB = B // BB

    idx = (sentences.astype(jnp.int32)
           .reshape(NB, BB, T).transpose(0, 2, 1).reshape(NB, T * BB))

    kern = functools.partial(_tagger_kernel, seq_len=T, hidden_dim=H,
                             block_b=BB, n_tags=_TAGSET)
    grid_spec = pltpu.PrefetchScalarGridSpec(
        num_scalar_prefetch=1,
        grid=(NB,),
        in_specs=[
            pl.BlockSpec(memory_space=pl.ANY),
            pl.BlockSpec((H, 4 * H), lambda nb, idx: (0, 0)),
            pl.BlockSpec((H, VPAD), lambda nb, idx: (0, 0)),
            pl.BlockSpec((1, VPAD), lambda nb, idx: (0, 0)),
        ],
        out_specs=pl.BlockSpec((None, BB * T, _TAGSET),
                               lambda nb, idx: (nb, 0, 0)),
        scratch_shapes=[
            pltpu.VMEM((T * BB, 4 * H), jnp.float32),
            pltpu.SemaphoreType.DMA((4 if T == 32 else T // min(_WC, T),)),
        ],
    )
    out = pl.pallas_call(
        kern,
        out_shape=jax.ShapeDtypeStruct((NB, BB * T, _TAGSET), jnp.float32),
        grid_spec=grid_spec,
        compiler_params=pltpu.CompilerParams(
            dimension_semantics=("parallel",),
            disable_bounds_checks=True),
    )(idx, xg_table, whh, wout, bout)

    return out.reshape(B, T, _TAGSET)


# speedup vs baseline: 1.0139x; 1.0139x over previous
import functools

import jax
import jax.numpy as jnp
from jax import lax
from jax.experimental import pallas as pl
from jax.experimental.pallas import tpu as pltpu

_TAGSET = 45
_BB = 128
_UNROLL = 16
_WC = 16


def _tagger_kernel(idx_ref, xg_tab_ref, whh_ref, wout_ref, bout_ref,
                   out_ref, xg_vmem, sems, *, seq_len, hidden_dim,
                   block_b, n_tags):
    T, H, BB = seq_len, hidden_dim, block_b
    wc = min(_WC, T)
    starts = [0, 16, 26] if T == 32 else list(range(0, T, wc))
    sizes = [b - a for a, b in zip(starts, starts[1:] + [T])]
    nb = pl.program_id(0)
    rows = T * BB

    def issue(k, carry):
        base = k * _UNROLL
        s = base * 0
        for b in starts[1:]:
            s = s + (base >= b * BB).astype(base.dtype)
        for u in range(_UNROLL):
            j = base + u
            pltpu.make_async_copy(
                xg_tab_ref.at[pl.ds(idx_ref[nb, j], 1), :],
                xg_vmem.at[pl.ds(j, 1), :],
                sems.at[s]).start()
        return carry

    lax.fori_loop(0, rows // _UNROLL, issue, 0)

    whh = whh_ref[...]
    wout = wout_ref[...]
    bout = bout_ref[...]

    h = jnp.zeros((BB, H), jnp.float32)
    c = jnp.zeros((BB, H), jnp.float32)
    for t in range(T):
        if t in starts:
            ci = starts.index(t)
            pltpu.make_async_copy(
                xg_tab_ref.at[pl.ds(0, BB * sizes[ci]), :],
                xg_vmem.at[pl.ds(t * BB, BB * sizes[ci]), :],
                sems.at[ci]).wait()
        xg_t = xg_vmem[pl.ds(t * BB, BB), :]
        if t == 0:
            gates = xg_t
        else:
            gates = xg_t + jnp.dot(h, whh,
                                   preferred_element_type=jnp.float32)
        th = jnp.tanh(0.5 * gates)
        i_g = 0.5 * th[:, 0 * H:1 * H] + 0.5
        f_g = 0.5 * th[:, 1 * H:2 * H] + 0.5
        g_g = th[:, 2 * H:3 * H]
        o_g = 0.5 * th[:, 3 * H:4 * H] + 0.5
        c = f_g * c + i_g * g_g
        h = o_g * jnp.tanh(c)
        logits = jnp.dot(h, wout, preferred_element_type=jnp.float32) + bout
        m = jnp.max(logits, axis=1, keepdims=True)
        z = logits - m
        lse = jnp.log(jnp.sum(jnp.exp(z), axis=1, keepdims=True))
        res = (z - lse)[:, :n_tags]
        out_ref[t:t + BB * T:T, :] = res


def kernel(sentences, xg_table, whh, wout, bout):
    B, T = sentences.shape
    H = whh.shape[0]
    VPAD = wout.shape[1]
    BB = _BB if B % _BB == 0 else B
    NB = B // BB

    idx = (sentences.astype(jnp.int32)
           .reshape(NB, BB, T).transpose(0, 2, 1).reshape(NB, T * BB))

    kern = functools.partial(_tagger_kernel, seq_len=T, hidden_dim=H,
                             block_b=BB, n_tags=_TAGSET)
    grid_spec = pltpu.PrefetchScalarGridSpec(
        num_scalar_prefetch=1,
        grid=(NB,),
        in_specs=[
            pl.BlockSpec(memory_space=pl.ANY),
            pl.BlockSpec((H, 4 * H), lambda nb, idx: (0, 0)),
            pl.BlockSpec((H, VPAD), lambda nb, idx: (0, 0)),
            pl.BlockSpec((1, VPAD), lambda nb, idx: (0, 0)),
        ],
        out_specs=pl.BlockSpec((None, BB * T, _TAGSET),
                               lambda nb, idx: (nb, 0, 0)),
        scratch_shapes=[
            pltpu.VMEM((T * BB, 4 * H), jnp.float32),
            pltpu.SemaphoreType.DMA((3 if T == 32 else T // min(_WC, T),)),
        ],
    )
    out = pl.pallas_call(
        kern,
        out_shape=jax.ShapeDtypeStruct((NB, BB * T, _TAGSET), jnp.float32),
        grid_spec=grid_spec,
        compiler_params=pltpu.CompilerParams(
            dimension_semantics=("parallel",),
            disable_bounds_checks=True),
    )(idx, xg_table, whh, wout, bout)

    return out.reshape(B, T, _TAGSET)


# speedup vs baseline: 1.0171x; 1.0032x over previous
import functools

import jax
import jax.numpy as jnp
from jax import lax
from jax.experimental import pallas as pl
from jax.experimental.pallas import tpu as pltpu

_TAGSET = 45
_BB = 128
_UNROLL = 16
_WC = 16


def _chunk_starts(T):
    if T == 32:
        return [0, 16, 26]
    return list(range(0, T, min(_WC, T)))


def _tagger_kernel(idx_ref, xg_tab_ref, whh_ref, wout_ref, bout_ref,
                   out_ref, xg_vmem, sems, *, seq_len, hidden_dim,
                   block_b, n_tags):
    T, H, BB = seq_len, hidden_dim, block_b
    starts = _chunk_starts(T)
    sizes = [b - a for a, b in zip(starts, starts[1:] + [T])]
    nb = pl.program_id(0)
    rows = T * BB

    def issue(k, carry):
        base = k * _UNROLL
        s = base * 0
        for b in starts[1:]:
            s = s + (base >= b * BB).astype(base.dtype)
        for u in range(_UNROLL):
            j = base + u
            pltpu.make_async_copy(
                xg_tab_ref.at[pl.ds(idx_ref[nb, j], 1), :],
                xg_vmem.at[pl.ds(j, 1), :],
                sems.at[s]).start()
        return carry

    lax.fori_loop(0, rows // _UNROLL, issue, 0)

    whh = whh_ref[...]
    wout = wout_ref[...]
    bout = bout_ref[...]

    h = jnp.zeros((BB, H), jnp.float32)
    c = jnp.zeros((BB, H), jnp.float32)
    for t in range(T):
        if t in starts:
            ci = starts.index(t)
            pltpu.make_async_copy(
                xg_tab_ref.at[pl.ds(0, BB * sizes[ci]), :],
                xg_vmem.at[pl.ds(t * BB, BB * sizes[ci]), :],
                sems.at[ci]).wait()
        xg_t = xg_vmem[pl.ds(t * BB, BB), :]
        if t == 0:
            gates = xg_t
        else:
            gates = xg_t + jnp.dot(h, whh,
                                   preferred_element_type=jnp.float32)
        th = jnp.tanh(0.5 * gates)
        i_g = 0.5 * th[:, 0 * H:1 * H] + 0.5
        f_g = 0.5 * th[:, 1 * H:2 * H] + 0.5
        g_g = th[:, 2 * H:3 * H]
        o_g = 0.5 * th[:, 3 * H:4 * H] + 0.5
        c = f_g * c + i_g * g_g
        h = o_g * jnp.tanh(c)
        logits = jnp.dot(h, wout, preferred_element_type=jnp.float32) + bout
        m = jnp.max(logits, axis=1, keepdims=True)
        z = logits - m
        lse = jnp.log(jnp.sum(jnp.exp(z), axis=1, keepdims=True))
        res = (z - lse)[:, :n_tags]
        out_ref[t:t + BB * T:T, :] = res


def kernel(sentences, xg_table, whh, wout, bout):
    B, T = sentences.shape
    H = whh.shape[0]
    VPAD = wout.shape[1]
    BB = _BB if B % _BB == 0 else B
    NB = B // BB

    idx = (sentences.astype(jnp.int32)
           .reshape(NB, BB, T).transpose(0, 2, 1).reshape(NB, T * BB))

    kern = functools.partial(_tagger_kernel, seq_len=T, hidden_dim=H,
                             block_b=BB, n_tags=_TAGSET)
    grid_spec = pltpu.PrefetchScalarGridSpec(
        num_scalar_prefetch=1,
        grid=(NB,),
        in_specs=[
            pl.BlockSpec(memory_space=pl.ANY),
            pl.BlockSpec((H, 4 * H), lambda nb, idx: (0, 0)),
            pl.BlockSpec((H, VPAD), lambda nb, idx: (0, 0)),
            pl.BlockSpec((1, VPAD), lambda nb, idx: (0, 0)),
        ],
        out_specs=pl.BlockSpec((None, BB * T, _TAGSET),
                               lambda nb, idx: (nb, 0, 0)),
        scratch_shapes=[
            pltpu.VMEM((T * BB, 4 * H), jnp.float32),
            pltpu.SemaphoreType.DMA((len(_chunk_starts(T)),)),
        ],
    )
    out = pl.pallas_call(
        kern,
        out_shape=jax.ShapeDtypeStruct((NB, BB * T, _TAGSET), jnp.float32),
        grid_spec=grid_spec,
        compiler_params=pltpu.CompilerParams(
            dimension_semantics=("parallel",),
            disable_bounds_checks=True),
    )(idx, xg_table, whh, wout, bout)

    return out.reshape(B, T, _TAGSET)
